# baseline (speedup 1.0000x reference)
import functools
import jax
import jax.numpy as jnp
from jax import lax
from jax.experimental import pallas as pl
from jax.experimental.pallas import tpu as pltpu
from jax.experimental.pallas import tpu_sc as plsc

_B, _LEN, _D, _V = 4096, 200, 64, 1000000
_NC, _NS = 2, 16
_NW = _NC * _NS
_BPW = _B // _NW
_RPW = _BPW * _LEN
_CH = _LEN
_UNITS = ((0, 104), (104, 96))
_NCHUNK = _RPW // _CH


def _sig_body(y_ref, w_ref, b_ref, o_ref):
    o_ref[...] = (
        jnp.dot(y_ref[...], w_ref[...].T, preferred_element_type=jnp.float32)
        + b_ref[...]
    )


def _compute_sig(y, w, bias):
    return pl.pallas_call(
        _sig_body,
        out_shape=jax.ShapeDtypeStruct((_B, _D), jnp.float32),
    )(y, w, bias.reshape(1, _D))


@functools.partial(
    pl.kernel,
    out_type=jax.ShapeDtypeStruct((_B, _LEN, _D), jnp.float32),
    mesh=plsc.VectorSubcoreMesh(
        core_axis_name="c", subcore_axis_name="s", num_cores=_NC, num_subcores=_NS
    ),
    scratch_types=[
        pltpu.VMEM((_RPW,), jnp.int32),
        pltpu.VMEM((_BPW * _D,), jnp.float32),
        pltpu.VMEM((_CH, _D), jnp.float32),
        pltpu.VMEM((_CH, _D), jnp.float32),
        pltpu.SemaphoreType.DMA,
        pltpu.SemaphoreType.DMA,
        pltpu.SemaphoreType.DMA,
        pltpu.SemaphoreType.DMA,
    ],
    compiler_params=pltpu.CompilerParams(use_tc_tiling_on_sc=False),
)
def _sc_embed(
    idx_hbm, sig_hbm, table_hbm, out_hbm,
    idx_v, sig_v, buf0, buf1, gsem0, gsem1, osem0, osem1,
):
    wid = lax.axis_index("s") * _NC + lax.axis_index("c")
    base_b = wid * _BPW

    pltpu.sync_copy(idx_hbm.at[pl.ds(wid * _RPW, _RPW)], idx_v)
    pltpu.sync_copy(sig_hbm.at[pl.ds(wid * _BPW * _D, _BPW * _D)], sig_v)

    def fire_gather(c, buf, gsem):
        for off, n in _UNITS:
            pltpu.async_copy(
                table_hbm.at[idx_v.at[pl.ds(c * _CH + off, n)]],
                buf.at[pl.ds(off, n)],
                gsem,
            )

    def wait_gather(c, buf, gsem):
        for off, n in _UNITS:
            pltpu.make_async_copy(
                table_hbm.at[idx_v.at[pl.ds(c * _CH + off, n)]],
                buf.at[pl.ds(off, n)],
                gsem,
            ).wait()

    def fire_write(c, buf, osem):
        pltpu.async_copy(buf, out_hbm.at[base_b + c], osem)

    def wait_write(buf, osem):
        pltpu.make_async_copy(buf, out_hbm.at[base_b], osem).wait()

    def add_sig(c, buf):
        svs = [
            sig_v[pl.ds(c * _D + 16 * d, 16)] for d in range(_D // 16)
        ]

        def row_body(r, carry):
            for d in range(_D // 16):
                plsc.addupdate(buf.at[r, pl.ds(16 * d, 16)], svs[d])
            return carry

        lax.fori_loop(0, _CH, row_body, 0, unroll=8)

    fire_gather(0, buf0, gsem0)

    def pair_body(i, carry):
        c0 = 2 * i
        c1 = c0 + 1

        @pl.when(i > 0)
        def _():
            wait_write(buf1, osem1)
        fire_gather(c1, buf1, gsem1)
        wait_gather(c0, buf0, gsem0)
        add_sig(c0, buf0)
        fire_write(c0, buf0, osem0)

        @pl.when(i < _NCHUNK // 2 - 1)
        def _():
            wait_write(buf0, osem0)
            fire_gather(c0 + 2, buf0, gsem0)
        wait_gather(c1, buf1, gsem1)
        add_sig(c1, buf1)
        fire_write(c1, buf1, osem1)
        return carry

    lax.fori_loop(0, _NCHUNK // 2, pair_body, 0)

    wait_write(buf0, osem0)
    wait_write(buf1, osem1)


@jax.jit
def kernel(x, y, embedding, W, b):
    sig = _compute_sig(y, W, b)
    return _sc_embed(x.reshape(_B * _LEN), sig.reshape(_B * _D), embedding)
